# baseline (speedup 1.0000x reference)
import jax
import jax.numpy as jnp
from jax import lax
from jax.experimental import pallas as pl
from jax.experimental.pallas import tpu as pltpu
from jax.experimental.pallas import tpu_sc as plsc

F32 = jnp.float32
NC, NS = 2, 16
NW = NC * NS
CH = 128
BLK = 1000
LEAK = 0.2
EPS_N = 1e-12


def _sc_edge_l1_body(src, dst, e32, h0, z128, z32,
                     g_out, es_out,
                     g_sp, es_sp, sidx, didx, rows, erows, sem):
    c = lax.axis_index("c")
    s = lax.axis_index("s")
    wid = s * NC + c
    n = g_sp.shape[0]
    r = n // NS
    pltpu.sync_copy(z128.at[pl.ds(s * r, r)], g_sp.at[pl.ds(s * r, r)])
    pltpu.sync_copy(z32.at[pl.ds(s * r, r)], es_sp.at[pl.ds(s * r, r)])
    plsc.subcore_barrier()
    cnt = src.shape[0] // CH // NW

    def body(i, carry):
        off = (wid + i * NW) * CH
        pltpu.sync_copy(src.at[pl.ds(off, CH)], sidx)
        pltpu.sync_copy(dst.at[pl.ds(off, CH)], didx)
        pltpu.sync_copy(e32.at[pl.ds(off, CH)], erows)
        pltpu.async_copy(h0.at[sidx], rows, sem).wait()
        pltpu.sync_copy(rows, g_sp.at[didx], add=True)
        pltpu.sync_copy(erows, es_sp.at[didx], add=True)
        return carry

    lax.fori_loop(0, cnt, body, 0)
    plsc.subcore_barrier()
    pltpu.sync_copy(g_sp.at[pl.ds(s * r, r)], g_out.at[c, pl.ds(s * r, r)])
    pltpu.sync_copy(es_sp.at[pl.ds(s * r, r)], es_out.at[c, pl.ds(s * r, r)])


def _sc_layer_body(src, dst, h, z128,
                   g_out,
                   g_sp, sidx0, didx0, rows0,
                   sidx1, didx1, rows1, sem0, sem1):
    c = lax.axis_index("c")
    s = lax.axis_index("s")
    wid = s * NC + c
    n = g_sp.shape[0]
    r = n // NS
    pltpu.sync_copy(z128.at[pl.ds(s * r, r)], g_sp.at[pl.ds(s * r, r)])
    plsc.subcore_barrier()
    cnt = src.shape[0] // CH // NW

    def body(i, carry):
        off0 = (wid + (2 * i) * NW) * CH
        off1 = (wid + (2 * i + 1) * NW) * CH
        pltpu.sync_copy(src.at[pl.ds(off0, CH)], sidx0)
        pltpu.sync_copy(src.at[pl.ds(off1, CH)], sidx1)
        a0 = pltpu.async_copy(h.at[sidx0], rows0, sem0)
        a1 = pltpu.async_copy(h.at[sidx1], rows1, sem1)
        pltpu.sync_copy(dst.at[pl.ds(off0, CH)], didx0)
        pltpu.sync_copy(dst.at[pl.ds(off1, CH)], didx1)
        a0.wait()
        pltpu.sync_copy(rows0, g_sp.at[didx0], add=True)
        a1.wait()
        pltpu.sync_copy(rows1, g_sp.at[didx1], add=True)
        return carry

    lax.fori_loop(0, cnt // 2, body, 0)
    plsc.subcore_barrier()
    pltpu.sync_copy(g_sp.at[pl.ds(s * r, r)], g_out.at[c, pl.ds(s * r, r)])


def _sc_mesh():
    return plsc.VectorSubcoreMesh(
        core_axis_name="c", subcore_axis_name="s",
        num_cores=NC, num_subcores=NS)


_SC_PARAMS = pltpu.CompilerParams(use_tc_tiling_on_sc=False)


def _make_sc_edge_l1(n, hid):
    return pl.kernel(
        _sc_edge_l1_body,
        out_type=(jax.ShapeDtypeStruct((NC, n, hid), F32),
                  jax.ShapeDtypeStruct((NC, n, 32), F32)),
        mesh=_sc_mesh(),
        compiler_params=_SC_PARAMS,
        scratch_types=[
            pltpu.VMEM_SHARED((n, hid), F32),
            pltpu.VMEM_SHARED((n, 32), F32),
            pltpu.VMEM((CH,), jnp.int32),
            pltpu.VMEM((CH,), jnp.int32),
            pltpu.VMEM((CH, hid), F32),
            pltpu.VMEM((CH, 32), F32),
            pltpu.SemaphoreType.DMA,
        ],
    )


def _make_sc_layer(n, hid):
    return pl.kernel(
        _sc_layer_body,
        out_type=jax.ShapeDtypeStruct((NC, n, hid), F32),
        mesh=_sc_mesh(),
        compiler_params=_SC_PARAMS,
        scratch_types=[
            pltpu.VMEM_SHARED((n, hid), F32),
            pltpu.VMEM((CH,), jnp.int32),
            pltpu.VMEM((CH,), jnp.int32),
            pltpu.VMEM((CH, hid), F32),
            pltpu.VMEM((CH,), jnp.int32),
            pltpu.VMEM((CH,), jnp.int32),
            pltpu.VMEM((CH, hid), F32),
            pltpu.SemaphoreType.DMA,
            pltpu.SemaphoreType.DMA,
        ],
    )


def _proj_body(x_ref, w_ref, b_ref, o_ref):
    o_ref[...] = (jnp.dot(x_ref[...], w_ref[...],
                          preferred_element_type=F32) + b_ref[...])


def _norm_lrelu(u):
    nrm = jnp.sqrt(jnp.sum(u * u, axis=-1, keepdims=True))
    upd = u / jnp.maximum(nrm, EPS_N)
    return jnp.where(upd >= 0, upd, LEAK * upd)


def _upd1_body(g_ref, es_ref, h0_ref, we_ref, be_ref,
               eagg_ref, h1_ref, hs_ref):
    ed = we_ref.shape[0]
    es_t = es_ref[0, :, :ed] + es_ref[1, :, :ed]
    deg_t = es_ref[0, :, ed:ed + 1] + es_ref[1, :, ed:ed + 1]
    eagg = (jnp.dot(es_t, we_ref[...], preferred_element_type=F32)
            + deg_t * be_ref[...])
    h1 = _norm_lrelu(g_ref[0] + g_ref[1] + eagg)
    eagg_ref[...] = eagg
    h1_ref[...] = h1
    hs_ref[...] = h0_ref[...] + h1


def _updk_body(g_ref, eagg_ref, hsp_ref, h_ref, hs_ref):
    hk = _norm_lrelu(g_ref[0] + g_ref[1] + eagg_ref[...])
    h_ref[...] = hk
    hs_ref[...] = hsp_ref[...] + hk


def _head_body(nrows, g_ref, eagg_ref, hsp_ref, c_ref,
               w1a_ref, w1b_ref, b1_ref, gam_ref, bet_ref, a_ref,
               w2_ref, b2_ref, o_ref, s1_ref, s2_ref):
    p = pl.program_id(0)
    i = pl.program_id(1)
    h3 = _norm_lrelu(g_ref[0] + g_ref[1] + eagg_ref[...])
    hs = hsp_ref[...] + h3
    z1 = (jnp.dot(hs, w1a_ref[...], preferred_element_type=F32)
          + jnp.dot(c_ref[...], w1b_ref[...], preferred_element_type=F32)
          + b1_ref[...])

    @pl.when(jnp.logical_and(p == 0, i == 0))
    def _():
        s1_ref[...] = jnp.zeros_like(s1_ref)
        s2_ref[...] = jnp.zeros_like(s2_ref)

    @pl.when(p == 0)
    def _():
        s1_ref[...] += jnp.sum(z1, axis=0, keepdims=True)
        s2_ref[...] += jnp.sum(z1 * z1, axis=0, keepdims=True)

    @pl.when(p == 1)
    def _():
        mu = s1_ref[...] * (1.0 / nrows)
        var = s2_ref[...] * (1.0 / nrows) - mu * mu
        zn = (z1 - mu) * lax.rsqrt(var + 1e-5) * gam_ref[...] + bet_ref[...]
        zp = jnp.where(zn >= 0, zn, a_ref[...] * zn)
        o_ref[...] = (jnp.dot(zp, w2_ref[...], preferred_element_type=F32)
                      + b2_ref[...])


def kernel(x, edge_index, e, c, W_node, b_node, W_edge, b_edge,
           W1, b1, gamma, beta, prelu_a, W2, b2):
    n, dn = x.shape
    ee, ed = e.shape
    hid = W_node.shape[1]
    hier = c.shape[1]
    h2d = W1.shape[1]
    ncls = W2.shape[1]
    nb = n // BLK

    npad = ((n + 127) // 128) * 128
    epad = ((ee + 2 * NW * CH - 1) // (2 * NW * CH)) * (2 * NW * CH)
    src = jnp.concatenate(
        [edge_index[0], jnp.zeros((epad - ee,), jnp.int32)])
    dst = jnp.concatenate(
        [edge_index[1],
         n + (jnp.arange(epad - ee, dtype=jnp.int32) % (npad - n))])
    e32 = jnp.concatenate(
        [e, jnp.ones((ee, 1), F32), jnp.zeros((ee, 32 - ed - 1), F32)],
        axis=1)
    e32 = jnp.concatenate([e32, jnp.zeros((epad - ee, 32), F32)])
    z128 = jnp.zeros((npad, hid), F32)
    z32 = jnp.zeros((npad, 32), F32)

    h0 = pl.pallas_call(
        _proj_body,
        grid=(nb,),
        in_specs=[pl.BlockSpec((BLK, dn), lambda i: (i, 0)),
                  pl.BlockSpec((dn, hid), lambda i: (0, 0)),
                  pl.BlockSpec((1, hid), lambda i: (0, 0))],
        out_specs=pl.BlockSpec((BLK, hid), lambda i: (i, 0)),
        out_shape=jax.ShapeDtypeStruct((n, hid), F32),
    )(x, W_node, b_node.reshape(1, hid))

    g1, es = _make_sc_edge_l1(npad, hid)(src, dst, e32, h0, z128, z32)

    eagg, h1, hs1 = pl.pallas_call(
        _upd1_body,
        grid=(nb,),
        in_specs=[pl.BlockSpec((NC, BLK, hid), lambda i: (0, i, 0)),
                  pl.BlockSpec((NC, BLK, 32), lambda i: (0, i, 0)),
                  pl.BlockSpec((BLK, hid), lambda i: (i, 0)),
                  pl.BlockSpec((ed, hid), lambda i: (0, 0)),
                  pl.BlockSpec((1, hid), lambda i: (0, 0))],
        out_specs=[pl.BlockSpec((BLK, hid), lambda i: (i, 0))] * 3,
        out_shape=[jax.ShapeDtypeStruct((n, hid), F32)] * 3,
    )(g1, es, h0, W_edge, b_edge.reshape(1, hid))

    sc_layer = _make_sc_layer(npad, hid)

    g2 = sc_layer(src, dst, h1, z128)
    h2, hs2 = pl.pallas_call(
        _updk_body,
        grid=(nb,),
        in_specs=[pl.BlockSpec((NC, BLK, hid), lambda i: (0, i, 0)),
                  pl.BlockSpec((BLK, hid), lambda i: (i, 0)),
                  pl.BlockSpec((BLK, hid), lambda i: (i, 0))],
        out_specs=[pl.BlockSpec((BLK, hid), lambda i: (i, 0))] * 2,
        out_shape=[jax.ShapeDtypeStruct((n, hid), F32)] * 2,
    )(g2, eagg, hs1)

    g3 = sc_layer(src, dst, h2, z128)

    import functools
    out = pl.pallas_call(
        functools.partial(_head_body, float(n)),
        grid=(2, nb),
        in_specs=[pl.BlockSpec((NC, BLK, hid), lambda p, i: (0, i, 0)),
                  pl.BlockSpec((BLK, hid), lambda p, i: (i, 0)),
                  pl.BlockSpec((BLK, hid), lambda p, i: (i, 0)),
                  pl.BlockSpec((BLK, hier), lambda p, i: (i, 0)),
                  pl.BlockSpec((hid, h2d), lambda p, i: (0, 0)),
                  pl.BlockSpec((hier, h2d), lambda p, i: (0, 0)),
                  pl.BlockSpec((1, h2d), lambda p, i: (0, 0)),
                  pl.BlockSpec((1, h2d), lambda p, i: (0, 0)),
                  pl.BlockSpec((1, h2d), lambda p, i: (0, 0)),
                  pl.BlockSpec((1, h2d), lambda p, i: (0, 0)),
                  pl.BlockSpec((h2d, ncls), lambda p, i: (0, 0)),
                  pl.BlockSpec((1, ncls), lambda p, i: (0, 0))],
        out_specs=pl.BlockSpec((BLK, ncls), lambda p, i: (i, 0)),
        out_shape=jax.ShapeDtypeStruct((n, ncls), F32),
        scratch_shapes=[pltpu.VMEM((1, h2d), F32),
                        pltpu.VMEM((1, h2d), F32)],
    )(g3, eagg, hs2, c,
      W1[:hid], W1[hid:], b1.reshape(1, h2d),
      gamma.reshape(1, h2d), beta.reshape(1, h2d),
      jnp.broadcast_to(prelu_a.reshape(1, 1), (1, h2d)),
      W2, b2.reshape(1, ncls))
    return out


# speedup vs baseline: 1.3132x; 1.3132x over previous
import jax
import jax.numpy as jnp
from jax import lax
from jax.experimental import pallas as pl
from jax.experimental.pallas import tpu as pltpu
from jax.experimental.pallas import tpu_sc as plsc

F32 = jnp.float32
NC, NS = 2, 16
NW = NC * NS
CH = 128
BLK = 1000
LEAK = 0.2
EPS_N = 1e-12


def _sc_edge_l1_body(sd, e32, h0, z128, z32,
                     g_out, es_out,
                     g_sp, es_sp, idx, rows, erows, sem):
    c = lax.axis_index("c")
    s = lax.axis_index("s")
    wid = s * NC + c
    n = g_sp.shape[0]
    r = n // NS
    pltpu.sync_copy(z128.at[pl.ds(s * r, r)], g_sp.at[pl.ds(s * r, r)])
    pltpu.sync_copy(z32.at[pl.ds(s * r, r)], es_sp.at[pl.ds(s * r, r)])
    plsc.subcore_barrier()
    cnt = sd.shape[0] // NW

    def body(i, carry):
        j = wid + i * NW
        pltpu.sync_copy(sd.at[j], idx)
        pltpu.sync_copy(e32.at[pl.ds(j * CH, CH)], erows)
        pltpu.async_copy(h0.at[idx.at[0]], rows, sem).wait()
        pltpu.sync_copy(rows, g_sp.at[idx.at[1]], add=True)
        pltpu.sync_copy(erows, es_sp.at[idx.at[1]], add=True)
        return carry

    lax.fori_loop(0, cnt, body, 0)
    plsc.subcore_barrier()
    pltpu.sync_copy(g_sp.at[pl.ds(s * r, r)], g_out.at[c, pl.ds(s * r, r)])
    pltpu.sync_copy(es_sp.at[pl.ds(s * r, r)], es_out.at[c, pl.ds(s * r, r)])


def _sc_layer_body(sd, h, z128,
                   g_out,
                   g_sp, idx, rows, sem):
    c = lax.axis_index("c")
    s = lax.axis_index("s")
    wid = s * NC + c
    n = g_sp.shape[0]
    r = n // NS
    pltpu.sync_copy(z128.at[pl.ds(s * r, r)], g_sp.at[pl.ds(s * r, r)])
    plsc.subcore_barrier()
    cnt = sd.shape[0] // NW

    def body(i, carry):
        pltpu.sync_copy(sd.at[wid + i * NW], idx)
        pltpu.async_copy(h.at[idx.at[0]], rows, sem).wait()
        pltpu.sync_copy(rows, g_sp.at[idx.at[1]], add=True)
        return carry

    lax.fori_loop(0, cnt, body, 0)
    plsc.subcore_barrier()
    pltpu.sync_copy(g_sp.at[pl.ds(s * r, r)], g_out.at[c, pl.ds(s * r, r)])


def _sc_mesh():
    return plsc.VectorSubcoreMesh(
        core_axis_name="c", subcore_axis_name="s",
        num_cores=NC, num_subcores=NS)


_SC_PARAMS = pltpu.CompilerParams(use_tc_tiling_on_sc=False)


def _make_sc_edge_l1(n, hid):
    return pl.kernel(
        _sc_edge_l1_body,
        out_type=(jax.ShapeDtypeStruct((NC, n, hid), F32),
                  jax.ShapeDtypeStruct((NC, n, 32), F32)),
        mesh=_sc_mesh(),
        compiler_params=_SC_PARAMS,
        scratch_types=[
            pltpu.VMEM_SHARED((n, hid), F32),
            pltpu.VMEM_SHARED((n, 32), F32),
            pltpu.VMEM((2, CH), jnp.int32),
            pltpu.VMEM((CH, hid), F32),
            pltpu.VMEM((CH, 32), F32),
            pltpu.SemaphoreType.DMA,
        ],
    )


def _make_sc_layer(n, hid):
    return pl.kernel(
        _sc_layer_body,
        out_type=jax.ShapeDtypeStruct((NC, n, hid), F32),
        mesh=_sc_mesh(),
        compiler_params=_SC_PARAMS,
        scratch_types=[
            pltpu.VMEM_SHARED((n, hid), F32),
            pltpu.VMEM((2, CH), jnp.int32),
            pltpu.VMEM((CH, hid), F32),
            pltpu.SemaphoreType.DMA,
        ],
    )


def _proj_body(x_ref, w_ref, b_ref, o_ref):
    o_ref[...] = (jnp.dot(x_ref[...], w_ref[...],
                          preferred_element_type=F32) + b_ref[...])


def _norm_lrelu(u):
    nrm = jnp.sqrt(jnp.sum(u * u, axis=-1, keepdims=True))
    upd = u / jnp.maximum(nrm, EPS_N)
    return jnp.where(upd >= 0, upd, LEAK * upd)


def _upd1_body(g_ref, es_ref, h0_ref, we_ref, be_ref,
               eagg_ref, h1_ref, hs_ref):
    ed = we_ref.shape[0]
    es_t = es_ref[0, :, :ed] + es_ref[1, :, :ed]
    deg_t = es_ref[0, :, ed:ed + 1] + es_ref[1, :, ed:ed + 1]
    eagg = (jnp.dot(es_t, we_ref[...], preferred_element_type=F32)
            + deg_t * be_ref[...])
    h1 = _norm_lrelu(g_ref[0] + g_ref[1] + eagg)
    eagg_ref[...] = eagg
    h1_ref[...] = h1
    hs_ref[...] = h0_ref[...] + h1


def _updk_body(g_ref, eagg_ref, hsp_ref, h_ref, hs_ref):
    hk = _norm_lrelu(g_ref[0] + g_ref[1] + eagg_ref[...])
    h_ref[...] = hk
    hs_ref[...] = hsp_ref[...] + hk


def _head_body(nrows, g_ref, eagg_ref, hsp_ref, c_ref,
               w1a_ref, w1b_ref, b1_ref, gam_ref, bet_ref, a_ref,
               w2_ref, b2_ref, o_ref, s1_ref, s2_ref):
    p = pl.program_id(0)
    i = pl.program_id(1)
    h3 = _norm_lrelu(g_ref[0] + g_ref[1] + eagg_ref[...])
    hs = hsp_ref[...] + h3
    z1 = (jnp.dot(hs, w1a_ref[...], preferred_element_type=F32)
          + jnp.dot(c_ref[...], w1b_ref[...], preferred_element_type=F32)
          + b1_ref[...])

    @pl.when(jnp.logical_and(p == 0, i == 0))
    def _():
        s1_ref[...] = jnp.zeros_like(s1_ref)
        s2_ref[...] = jnp.zeros_like(s2_ref)

    @pl.when(p == 0)
    def _():
        s1_ref[...] += jnp.sum(z1, axis=0, keepdims=True)
        s2_ref[...] += jnp.sum(z1 * z1, axis=0, keepdims=True)

    @pl.when(p == 1)
    def _():
        mu = s1_ref[...] * (1.0 / nrows)
        var = s2_ref[...] * (1.0 / nrows) - mu * mu
        zn = (z1 - mu) * lax.rsqrt(var + 1e-5) * gam_ref[...] + bet_ref[...]
        zp = jnp.where(zn >= 0, zn, a_ref[...] * zn)
        o_ref[...] = (jnp.dot(zp, w2_ref[...], preferred_element_type=F32)
                      + b2_ref[...])


def kernel(x, edge_index, e, c, W_node, b_node, W_edge, b_edge,
           W1, b1, gamma, beta, prelu_a, W2, b2):
    n, dn = x.shape
    ee, ed = e.shape
    hid = W_node.shape[1]
    hier = c.shape[1]
    h2d = W1.shape[1]
    ncls = W2.shape[1]
    nb = n // BLK

    npad = ((n + 127) // 128) * 128
    epad = ((ee + NW * CH - 1) // (NW * CH)) * (NW * CH)
    src = jnp.concatenate(
        [edge_index[0], jnp.zeros((epad - ee,), jnp.int32)])
    dst = jnp.concatenate(
        [edge_index[1],
         n + (jnp.arange(epad - ee, dtype=jnp.int32) % (npad - n))])
    e32 = jnp.concatenate(
        [e, jnp.ones((ee, 1), F32), jnp.zeros((ee, 32 - ed - 1), F32)],
        axis=1)
    e32 = jnp.concatenate([e32, jnp.zeros((epad - ee, 32), F32)])
    sd = jnp.stack([src.reshape(-1, CH), dst.reshape(-1, CH)], axis=1)
    z128 = jnp.zeros((npad, hid), F32)
    z32 = jnp.zeros((npad, 32), F32)

    h0 = pl.pallas_call(
        _proj_body,
        grid=(nb,),
        in_specs=[pl.BlockSpec((BLK, dn), lambda i: (i, 0)),
                  pl.BlockSpec((dn, hid), lambda i: (0, 0)),
                  pl.BlockSpec((1, hid), lambda i: (0, 0))],
        out_specs=pl.BlockSpec((BLK, hid), lambda i: (i, 0)),
        out_shape=jax.ShapeDtypeStruct((n, hid), F32),
    )(x, W_node, b_node.reshape(1, hid))

    g1, es = _make_sc_edge_l1(npad, hid)(sd, e32, h0, z128, z32)

    eagg, h1, hs1 = pl.pallas_call(
        _upd1_body,
        grid=(nb,),
        in_specs=[pl.BlockSpec((NC, BLK, hid), lambda i: (0, i, 0)),
                  pl.BlockSpec((NC, BLK, 32), lambda i: (0, i, 0)),
                  pl.BlockSpec((BLK, hid), lambda i: (i, 0)),
                  pl.BlockSpec((ed, hid), lambda i: (0, 0)),
                  pl.BlockSpec((1, hid), lambda i: (0, 0))],
        out_specs=[pl.BlockSpec((BLK, hid), lambda i: (i, 0))] * 3,
        out_shape=[jax.ShapeDtypeStruct((n, hid), F32)] * 3,
    )(g1, es, h0, W_edge, b_edge.reshape(1, hid))

    sc_layer = _make_sc_layer(npad, hid)

    g2 = sc_layer(sd, h1, z128)
    h2, hs2 = pl.pallas_call(
        _updk_body,
        grid=(nb,),
        in_specs=[pl.BlockSpec((NC, BLK, hid), lambda i: (0, i, 0)),
                  pl.BlockSpec((BLK, hid), lambda i: (i, 0)),
                  pl.BlockSpec((BLK, hid), lambda i: (i, 0))],
        out_specs=[pl.BlockSpec((BLK, hid), lambda i: (i, 0))] * 2,
        out_shape=[jax.ShapeDtypeStruct((n, hid), F32)] * 2,
    )(g2, eagg, hs1)

    g3 = sc_layer(sd, h2, z128)

    import functools
    out = pl.pallas_call(
        functools.partial(_head_body, float(n)),
        grid=(2, nb),
        in_specs=[pl.BlockSpec((NC, BLK, hid), lambda p, i: (0, i, 0)),
                  pl.BlockSpec((BLK, hid), lambda p, i: (i, 0)),
                  pl.BlockSpec((BLK, hid), lambda p, i: (i, 0)),
                  pl.BlockSpec((BLK, hier), lambda p, i: (i, 0)),
                  pl.BlockSpec((hid, h2d), lambda p, i: (0, 0)),
                  pl.BlockSpec((hier, h2d), lambda p, i: (0, 0)),
                  pl.BlockSpec((1, h2d), lambda p, i: (0, 0)),
                  pl.BlockSpec((1, h2d), lambda p, i: (0, 0)),
                  pl.BlockSpec((1, h2d), lambda p, i: (0, 0)),
                  pl.BlockSpec((1, h2d), lambda p, i: (0, 0)),
                  pl.BlockSpec((h2d, ncls), lambda p, i: (0, 0)),
                  pl.BlockSpec((1, ncls), lambda p, i: (0, 0))],
        out_specs=pl.BlockSpec((BLK, ncls), lambda p, i: (i, 0)),
        out_shape=jax.ShapeDtypeStruct((n, ncls), F32),
        scratch_shapes=[pltpu.VMEM((1, h2d), F32),
                        pltpu.VMEM((1, h2d), F32)],
    )(g3, eagg, hs2, c,
      W1[:hid], W1[hid:], b1.reshape(1, h2d),
      gamma.reshape(1, h2d), beta.reshape(1, h2d),
      jnp.broadcast_to(prelu_a.reshape(1, 1), (1, h2d)),
      W2, b2.reshape(1, ncls))
    return out


# speedup vs baseline: 1.3219x; 1.0066x over previous
import jax
import jax.numpy as jnp
from jax import lax
from jax.experimental import pallas as pl
from jax.experimental.pallas import tpu as pltpu
from jax.experimental.pallas import tpu_sc as plsc

F32 = jnp.float32
NC, NS = 2, 16
NW = NC * NS
CH = 128
BLK = 1000
LEAK = 0.2
EPS_N = 1e-12


def _sc_edge_l1_body(sd, e32, h0, z128, z32,
                     g_out, es_out,
                     g_sp, es_sp, idx, rows, erows, sem):
    c = lax.axis_index("c")
    s = lax.axis_index("s")
    wid = s * NC + c
    n = g_sp.shape[0]
    r = n // NS
    pltpu.sync_copy(z128.at[pl.ds(s * r, r)], g_sp.at[pl.ds(s * r, r)])
    pltpu.sync_copy(z32.at[pl.ds(s * r, r)], es_sp.at[pl.ds(s * r, r)])
    plsc.subcore_barrier()
    cnt = sd.shape[0] // NW

    def body(i, carry):
        j = wid + i * NW
        pltpu.sync_copy(sd.at[j], idx)
        pltpu.sync_copy(e32.at[pl.ds(j * CH, CH)], erows)
        pltpu.async_copy(h0.at[idx.at[0]], rows, sem).wait()
        pltpu.sync_copy(rows, g_sp.at[idx.at[1]], add=True)
        pltpu.sync_copy(erows, es_sp.at[idx.at[1]], add=True)
        return carry

    lax.fori_loop(0, cnt, body, 0)
    plsc.subcore_barrier()
    pltpu.sync_copy(g_sp.at[pl.ds(s * r, r)], g_out.at[c, pl.ds(s * r, r)])
    pltpu.sync_copy(es_sp.at[pl.ds(s * r, r)], es_out.at[c, pl.ds(s * r, r)])


def _sc_layer_body(sd, h, z128,
                   g_out,
                   g_sp, idx, rows, sem):
    c = lax.axis_index("c")
    s = lax.axis_index("s")
    wid = s * NC + c
    n = g_sp.shape[0]
    r = n // NS
    pltpu.sync_copy(z128.at[pl.ds(s * r, r)], g_sp.at[pl.ds(s * r, r)])
    plsc.subcore_barrier()
    cnt = sd.shape[0] // NW

    def body(i, carry):
        pltpu.sync_copy(sd.at[wid + i * NW], idx)
        pltpu.async_copy(h.at[idx.at[0]], rows, sem).wait()
        pltpu.sync_copy(rows, g_sp.at[idx.at[1]], add=True)
        return carry

    lax.fori_loop(0, cnt, body, 0)
    plsc.subcore_barrier()
    pltpu.sync_copy(g_sp.at[pl.ds(s * r, r)], g_out.at[c, pl.ds(s * r, r)])


def _sc_mesh():
    return plsc.VectorSubcoreMesh(
        core_axis_name="c", subcore_axis_name="s",
        num_cores=NC, num_subcores=NS)


_SC_PARAMS = pltpu.CompilerParams(use_tc_tiling_on_sc=False)


def _make_sc_edge_l1(n, hid):
    return pl.kernel(
        _sc_edge_l1_body,
        out_type=(jax.ShapeDtypeStruct((NC, n, hid), F32),
                  jax.ShapeDtypeStruct((NC, n, 32), F32)),
        mesh=_sc_mesh(),
        compiler_params=_SC_PARAMS,
        scratch_types=[
            pltpu.VMEM_SHARED((n, hid), F32),
            pltpu.VMEM_SHARED((n, 32), F32),
            pltpu.VMEM((2, CH), jnp.int32),
            pltpu.VMEM((CH, hid), F32),
            pltpu.VMEM((CH, 32), F32),
            pltpu.SemaphoreType.DMA,
        ],
    )


def _make_sc_layer(n, hid):
    return pl.kernel(
        _sc_layer_body,
        out_type=jax.ShapeDtypeStruct((NC, n, hid), F32),
        mesh=_sc_mesh(),
        compiler_params=_SC_PARAMS,
        scratch_types=[
            pltpu.VMEM_SHARED((n, hid), F32),
            pltpu.VMEM((2, CH), jnp.int32),
            pltpu.VMEM((CH, hid), F32),
            pltpu.SemaphoreType.DMA,
        ],
    )


def _proj_body(x_ref, w_ref, b_ref, o_ref):
    o_ref[...] = (jnp.dot(x_ref[...], w_ref[...],
                          preferred_element_type=F32) + b_ref[...])


def _norm_lrelu(u):
    nrm = jnp.sqrt(jnp.sum(u * u, axis=-1, keepdims=True))
    upd = u / jnp.maximum(nrm, EPS_N)
    return jnp.where(upd >= 0, upd, LEAK * upd)


def _upd1_body(g_ref, es_ref, h0_ref, we_ref, be_ref,
               eagg_ref, h1_ref, hs_ref):
    ed = we_ref.shape[0]
    es_t = es_ref[0, :, :ed] + es_ref[1, :, :ed]
    deg_t = es_ref[0, :, ed:ed + 1] + es_ref[1, :, ed:ed + 1]
    eagg = (jnp.dot(es_t, we_ref[...], preferred_element_type=F32)
            + deg_t * be_ref[...])
    h1 = _norm_lrelu(g_ref[0] + g_ref[1] + eagg)
    eagg_ref[...] = eagg
    h1_ref[...] = h1
    hs_ref[...] = h0_ref[...] + h1


def _updk_body(g_ref, eagg_ref, hsp_ref, h_ref, hs_ref):
    hk = _norm_lrelu(g_ref[0] + g_ref[1] + eagg_ref[...])
    h_ref[...] = hk
    hs_ref[...] = hsp_ref[...] + hk


def _head_body(nrows, g_ref, eagg_ref, hsp_ref, c_ref,
               w1a_ref, w1b_ref, b1_ref, gam_ref, bet_ref, a_ref,
               w2_ref, b2_ref, o_ref, s1_ref, s2_ref, z_ref):
    p = pl.program_id(0)
    i = pl.program_id(1)
    blk = o_ref.shape[0]

    @pl.when(jnp.logical_and(p == 0, i == 0))
    def _():
        s1_ref[...] = jnp.zeros_like(s1_ref)
        s2_ref[...] = jnp.zeros_like(s2_ref)

    @pl.when(p == 0)
    def _():
        h3 = _norm_lrelu(g_ref[0] + g_ref[1] + eagg_ref[...])
        hs = hsp_ref[...] + h3
        z1 = (jnp.dot(hs, w1a_ref[...], preferred_element_type=F32)
              + jnp.dot(c_ref[...], w1b_ref[...], preferred_element_type=F32)
              + b1_ref[...])
        z_ref[pl.ds(i * blk, blk), :] = z1
        s1_ref[...] += jnp.sum(z1, axis=0, keepdims=True)
        s2_ref[...] += jnp.sum(z1 * z1, axis=0, keepdims=True)

    @pl.when(p == 1)
    def _():
        z1 = z_ref[pl.ds(i * blk, blk), :]
        mu = s1_ref[...] * (1.0 / nrows)
        var = s2_ref[...] * (1.0 / nrows) - mu * mu
        zn = (z1 - mu) * lax.rsqrt(var + 1e-5) * gam_ref[...] + bet_ref[...]
        zp = jnp.where(zn >= 0, zn, a_ref[...] * zn)
        o_ref[...] = (jnp.dot(zp, w2_ref[...], preferred_element_type=F32)
                      + b2_ref[...])


def kernel(x, edge_index, e, c, W_node, b_node, W_edge, b_edge,
           W1, b1, gamma, beta, prelu_a, W2, b2):
    n, dn = x.shape
    ee, ed = e.shape
    hid = W_node.shape[1]
    hier = c.shape[1]
    h2d = W1.shape[1]
    ncls = W2.shape[1]
    nb = n // BLK

    npad = ((n + 127) // 128) * 128
    epad = ((ee + NW * CH - 1) // (NW * CH)) * (NW * CH)
    src = jnp.concatenate(
        [edge_index[0], jnp.zeros((epad - ee,), jnp.int32)])
    dst = jnp.concatenate(
        [edge_index[1],
         n + (jnp.arange(epad - ee, dtype=jnp.int32) % (npad - n))])
    e32 = jnp.concatenate(
        [e, jnp.ones((ee, 1), F32), jnp.zeros((ee, 32 - ed - 1), F32)],
        axis=1)
    e32 = jnp.concatenate([e32, jnp.zeros((epad - ee, 32), F32)])
    sd = jnp.stack([src.reshape(-1, CH), dst.reshape(-1, CH)], axis=1)
    z128 = jnp.zeros((npad, hid), F32)
    z32 = jnp.zeros((npad, 32), F32)

    h0 = pl.pallas_call(
        _proj_body,
        grid=(nb,),
        in_specs=[pl.BlockSpec((BLK, dn), lambda i: (i, 0)),
                  pl.BlockSpec((dn, hid), lambda i: (0, 0)),
                  pl.BlockSpec((1, hid), lambda i: (0, 0))],
        out_specs=pl.BlockSpec((BLK, hid), lambda i: (i, 0)),
        out_shape=jax.ShapeDtypeStruct((n, hid), F32),
    )(x, W_node, b_node.reshape(1, hid))

    g1, es = _make_sc_edge_l1(npad, hid)(sd, e32, h0, z128, z32)

    eagg, h1, hs1 = pl.pallas_call(
        _upd1_body,
        grid=(nb,),
        in_specs=[pl.BlockSpec((NC, BLK, hid), lambda i: (0, i, 0)),
                  pl.BlockSpec((NC, BLK, 32), lambda i: (0, i, 0)),
                  pl.BlockSpec((BLK, hid), lambda i: (i, 0)),
                  pl.BlockSpec((ed, hid), lambda i: (0, 0)),
                  pl.BlockSpec((1, hid), lambda i: (0, 0))],
        out_specs=[pl.BlockSpec((BLK, hid), lambda i: (i, 0))] * 3,
        out_shape=[jax.ShapeDtypeStruct((n, hid), F32)] * 3,
    )(g1, es, h0, W_edge, b_edge.reshape(1, hid))

    sc_layer = _make_sc_layer(npad, hid)

    g2 = sc_layer(sd, h1, z128)
    h2, hs2 = pl.pallas_call(
        _updk_body,
        grid=(nb,),
        in_specs=[pl.BlockSpec((NC, BLK, hid), lambda i: (0, i, 0)),
                  pl.BlockSpec((BLK, hid), lambda i: (i, 0)),
                  pl.BlockSpec((BLK, hid), lambda i: (i, 0))],
        out_specs=[pl.BlockSpec((BLK, hid), lambda i: (i, 0))] * 2,
        out_shape=[jax.ShapeDtypeStruct((n, hid), F32)] * 2,
    )(g2, eagg, hs1)

    g3 = sc_layer(sd, h2, z128)

    import functools
    out = pl.pallas_call(
        functools.partial(_head_body, float(n)),
        grid=(2, nb),
        in_specs=[pl.BlockSpec((NC, BLK, hid), lambda p, i: (0, i * (1 - p), 0)),
                  pl.BlockSpec((BLK, hid), lambda p, i: (i * (1 - p), 0)),
                  pl.BlockSpec((BLK, hid), lambda p, i: (i * (1 - p), 0)),
                  pl.BlockSpec((BLK, hier), lambda p, i: (i * (1 - p), 0)),
                  pl.BlockSpec((hid, h2d), lambda p, i: (0, 0)),
                  pl.BlockSpec((hier, h2d), lambda p, i: (0, 0)),
                  pl.BlockSpec((1, h2d), lambda p, i: (0, 0)),
                  pl.BlockSpec((1, h2d), lambda p, i: (0, 0)),
                  pl.BlockSpec((1, h2d), lambda p, i: (0, 0)),
                  pl.BlockSpec((1, h2d), lambda p, i: (0, 0)),
                  pl.BlockSpec((h2d, ncls), lambda p, i: (0, 0)),
                  pl.BlockSpec((1, ncls), lambda p, i: (0, 0))],
        out_specs=pl.BlockSpec((BLK, ncls), lambda p, i: (i, 0)),
        out_shape=jax.ShapeDtypeStruct((n, ncls), F32),
        scratch_shapes=[pltpu.VMEM((1, h2d), F32),
                        pltpu.VMEM((1, h2d), F32),
                        pltpu.VMEM((n, h2d), F32)],
    )(g3, eagg, hs2, c,
      W1[:hid], W1[hid:], b1.reshape(1, h2d),
      gamma.reshape(1, h2d), beta.reshape(1, h2d),
      jnp.broadcast_to(prelu_a.reshape(1, 1), (1, h2d)),
      W2, b2.reshape(1, ncls))
    return out
